# baseline (speedup 1.0000x reference)
import functools

import jax
import jax.numpy as jnp
from jax import lax
from jax.experimental import pallas as pl
from jax.experimental.pallas import tpu as pltpu
from jax.experimental.pallas import tpu_sc as plsc

_NC = 2
_NS = 16
_NW = _NC * _NS


def _table_body(emb_ref, wt_ref, b_ref, out_ref):
    out_ref[...] = (
        jax.lax.dot_general(
            emb_ref[...],
            wt_ref[...],
            (((1,), (0,)), ((), ())),
            preferred_element_type=jnp.float32,
            precision=jax.lax.Precision.HIGHEST,
        )
        + b_ref[...]
    )


def _build_table_padded(emb, Wt, b2d, vp):
    v, _ = emb.shape
    return pl.pallas_call(
        _table_body,
        out_shape=jax.ShapeDtypeStruct((v, vp), jnp.float32),
    )(emb, Wt, b2d)


def _make_gather(vp, n_rows, chunk):
    per_w = n_rows // _NW
    n_chunks = per_w // chunk
    mesh = plsc.VectorSubcoreMesh(core_axis_name="c", subcore_axis_name="s")

    @functools.partial(
        pl.kernel,
        mesh=mesh,
        out_type=jax.ShapeDtypeStruct((n_rows, vp), jnp.float32),
        scratch_types=[
            pltpu.VMEM((per_w,), jnp.int32),
            pltpu.VMEM((chunk, vp), jnp.float32),
            pltpu.SemaphoreType.DMA,
        ],
    )
    def gather(table_hbm, idx_hbm, out_hbm, idx_v, rows_v, sem):
        wid = lax.axis_index("s") * _NC + lax.axis_index("c")
        base = wid * per_w
        pltpu.sync_copy(idx_hbm.at[pl.ds(base, per_w)], idx_v)

        def body(g, carry):
            off = g * chunk
            pltpu.async_copy(
                table_hbm.at[idx_v.at[pl.ds(off, chunk)]], rows_v, sem
            ).wait()
            pltpu.sync_copy(rows_v, out_hbm.at[pl.ds(base + off, chunk)])
            return carry

        lax.fori_loop(0, n_chunks, body, 0)

    return gather


def kernel(x, emb, W, b):
    bsz, seq = x.shape
    v, _ = emb.shape
    vp = (v + 127) // 128 * 128
    wt = jnp.pad(W.T, ((0, 0), (0, vp - v)))
    b2d = jnp.pad(b.reshape(1, v), ((0, 0), (0, vp - v)))
    table = _build_table_padded(emb, wt, b2d, vp)
    flat_idx = x.reshape(-1).astype(jnp.int32)
    out = _make_gather(vp, bsz * seq, 40)(table, flat_idx)
    return out[:, :v].reshape(bsz, seq, v)
